# Initial kernel scaffold; baseline (speedup 1.0000x reference)
import jax
import jax.numpy as jnp
from jax.experimental import pallas as pl


def kernel(x, Wl1, bl1, Wr1, br1, We1, att1, bias1, Wl2, bl2, Wr2, br2, We2, att2, bias2):
    raise NotImplementedError("write your pallas kernel here")


# speedup vs baseline: 1.4330x; 1.4330x over previous
import jax
import jax.numpy as jnp
from jax.experimental import pallas as pl

_B, _C, _H, _W = 2, 128, 256, 256
_PS = 16
_NH = _H // _PS
_NW = _W // _PS
_N = _NH * _NW
_HEADS = 8
_OUTC = _C // _HEADS
_THR = 0.5
_DT = 16
_NT = _N // _DT
_F32 = jnp.float32
_NEG = -1e30


def _pool_body(x_ref, o_ref):
    xb = x_ref[0]
    s = jnp.sum(xb, axis=1)
    w_ids = jax.lax.broadcasted_iota(jnp.int32, (_W, _NW), 0) // _PS
    p_ids = jax.lax.broadcasted_iota(jnp.int32, (_W, _NW), 1)
    pmat = jnp.where(w_ids == p_ids, 1.0 / (_PS * _PS), 0.0).astype(_F32)
    o_ref[0, 0] = jnp.dot(s, pmat, preferred_element_type=_F32)


def _adj_body(nf_ref, a_ref):
    xb = nf_ref[0]
    mu = jnp.mean(xb, axis=-1, keepdims=True)
    xc = xb - mu
    num = jax.lax.dot_general(xc, xc, (((1,), (1,)), ((), ())),
                              preferred_element_type=_F32)
    sq = jnp.sum(xc * xc, axis=-1, keepdims=True)
    nrm = jnp.sqrt(sq)
    outer = jax.lax.dot_general(nrm, nrm, (((1,), (1,)), ((), ())),
                                preferred_element_type=_F32)
    corr = num / (outer + 1e-8)
    a_ref[0] = (corr > _THR).astype(_F32)


def _gat_body(xin_ref, xt_ref, mh_ref, wl_ref, bl_ref, wr_ref, br_ref,
              we_ref, attc_ref, bias_ref, o_ref):
    xin = xin_ref[0]
    xt = xt_ref[0]
    xl = jnp.dot(xin, wl_ref[...], preferred_element_type=_F32) + bl_ref[...]
    xr = jnp.dot(xt, wr_ref[...], preferred_element_type=_F32) + br_ref[...]
    xre = xr + we_ref[...]
    pair = xl[None, :, :] + xre[:, None, :]
    e = jnp.where(pair >= 0, pair, 0.2 * pair)
    attc = attc_ref[...]
    h_ids = jax.lax.broadcasted_iota(jnp.int32, (_C, _HEADS), 0) // _OUTC
    c_ids = jax.lax.broadcasted_iota(jnp.int32, (_C, _HEADS), 1)
    attm = jnp.where(h_ids == c_ids, attc, 0.0)
    logits = jnp.dot(e.reshape(_DT * _N, _C), attm,
                     preferred_element_type=_F32).reshape(_DT, _N, _HEADS)
    logits = jnp.where(mh_ref[0] > 0.0, logits, _NEG)
    m = jnp.max(logits, axis=1, keepdims=True)
    p = jnp.exp(logits - m)
    alpha = p / jnp.sum(p, axis=1, keepdims=True)
    outs = []
    for h in range(_HEADS):
        a_h = alpha[:, :, h]
        xl_h = xl[:, h * _OUTC:(h + 1) * _OUTC]
        outs.append(jnp.dot(a_h, xl_h, preferred_element_type=_F32))
    out = jnp.concatenate(outs, axis=-1) + bias_ref[...]
    o_ref[0] = jnp.where(out > 0, out, jnp.exp(out) - 1.0)


def _add_body(x_ref, g_ref, o_ref):
    xb = x_ref[0]
    gr = g_ref[0, 0]
    p_ids = jax.lax.broadcasted_iota(jnp.int32, (_NW, _W), 0)
    w_ids = jax.lax.broadcasted_iota(jnp.int32, (_NW, _W), 1) // _PS
    emat = jnp.where(p_ids == w_ids, 1.0, 0.0).astype(_F32)
    wide = jnp.dot(gr, emat, preferred_element_type=_F32)
    o_ref[0] = xb + wide[:, None, :]


def _gat_layer(xin, maskh, Wl, bl, Wr, br, We, att, bias):
    return pl.pallas_call(
        _gat_body,
        grid=(_B, _NT),
        in_specs=[
            pl.BlockSpec((1, _N, _C), lambda b, t: (b, 0, 0)),
            pl.BlockSpec((1, _DT, _C), lambda b, t: (b, t, 0)),
            pl.BlockSpec((1, _DT, _N, _HEADS), lambda b, t: (b, t, 0, 0)),
            pl.BlockSpec((_C, _C), lambda b, t: (0, 0)),
            pl.BlockSpec((1, _C), lambda b, t: (0, 0)),
            pl.BlockSpec((_C, _C), lambda b, t: (0, 0)),
            pl.BlockSpec((1, _C), lambda b, t: (0, 0)),
            pl.BlockSpec((1, _C), lambda b, t: (0, 0)),
            pl.BlockSpec((_C, 1), lambda b, t: (0, 0)),
            pl.BlockSpec((1, _C), lambda b, t: (0, 0)),
        ],
        out_specs=pl.BlockSpec((1, _DT, _C), lambda b, t: (b, t, 0)),
        out_shape=jax.ShapeDtypeStruct((_B, _N, _C), _F32),
    )(xin, xin, maskh, Wl, bl.reshape(1, _C), Wr, br.reshape(1, _C),
      We.reshape(1, _C), att.reshape(_C, 1), bias.reshape(1, _C))


def kernel(x, Wl1, bl1, Wr1, br1, We1, att1, bias1,
           Wl2, bl2, Wr2, br2, We2, att2, bias2):
    pool = pl.pallas_call(
        _pool_body,
        grid=(_B, _NH),
        in_specs=[pl.BlockSpec((1, _C, _PS, _W), lambda b, h: (b, 0, h, 0))],
        out_specs=pl.BlockSpec((1, 1, _C, _NW), lambda b, h: (b, h, 0, 0)),
        out_shape=jax.ShapeDtypeStruct((_B, _NH, _C, _NW), _F32),
    )(x)
    nf = pool.transpose(0, 1, 3, 2).reshape(_B, _N, _C)

    adj = pl.pallas_call(
        _adj_body,
        grid=(_B,),
        in_specs=[pl.BlockSpec((1, _N, _C), lambda b: (b, 0, 0))],
        out_specs=pl.BlockSpec((1, _N, _N), lambda b: (b, 0, 0)),
        out_shape=jax.ShapeDtypeStruct((_B, _N, _N), _F32),
    )(nf)
    maskh = jnp.broadcast_to(adj[:, :, :, None], (_B, _N, _N, _HEADS))

    h1 = _gat_layer(nf, maskh, Wl1, bl1, Wr1, br1, We1, att1, bias1)
    h2 = _gat_layer(h1, maskh, Wl2, bl2, Wr2, br2, We2, att2, bias2)

    g = h2.reshape(_B, _NH, _NW, _C).transpose(0, 1, 3, 2)

    return pl.pallas_call(
        _add_body,
        grid=(_B, _NH),
        in_specs=[
            pl.BlockSpec((1, _C, _PS, _W), lambda b, h: (b, 0, h, 0)),
            pl.BlockSpec((1, 1, _C, _NW), lambda b, h: (b, h, 0, 0)),
        ],
        out_specs=pl.BlockSpec((1, _C, _PS, _W), lambda b, h: (b, 0, h, 0)),
        out_shape=jax.ShapeDtypeStruct((_B, _C, _H, _W), _F32),
    )(x, g)


# speedup vs baseline: 2.9471x; 2.0566x over previous
import jax
import jax.numpy as jnp
from jax.experimental import pallas as pl

_B, _C, _H, _W = 2, 128, 256, 256
_PS = 16
_NH = _H // _PS
_NW = _W // _PS
_N = _NH * _NW
_HEADS = 8
_OUTC = _C // _HEADS
_THR = 0.5
_DT = 16
_NT = _N // _DT
_F32 = jnp.float32


def _pool_body(x_ref, o_ref):
    xb = x_ref[0]
    s = jnp.sum(xb, axis=1)
    w_ids = jax.lax.broadcasted_iota(jnp.int32, (_W, _NW), 0) // _PS
    p_ids = jax.lax.broadcasted_iota(jnp.int32, (_W, _NW), 1)
    pmat = jnp.where(w_ids == p_ids, 1.0 / (_PS * _PS), 0.0).astype(_F32)
    o_ref[0, 0] = jnp.dot(s, pmat, preferred_element_type=_F32)


def _adj_body(nf_ref, a_ref):
    xb = nf_ref[0]
    mu = jnp.mean(xb, axis=-1, keepdims=True)
    xc = xb - mu
    num = jax.lax.dot_general(xc, xc, (((1,), (1,)), ((), ())),
                              preferred_element_type=_F32)
    sq = jnp.sum(xc * xc, axis=-1, keepdims=True)
    nrm = jnp.sqrt(sq)
    outer = jax.lax.dot_general(nrm, nrm, (((1,), (1,)), ((), ())),
                                preferred_element_type=_F32)
    corr = num / (outer + 1e-8)
    a_ref[0] = (corr > _THR).astype(_F32)


def _gat_body(xin_ref, xt_ref, a_ref, wl_ref, blc_ref, blr_ref, wr_ref,
              brwe_ref, attr_ref, bias_ref, o_ref):
    xin = xin_ref[0]
    xt = xt_ref[0]
    xlT = jax.lax.dot_general(wl_ref[...], xin, (((0,), (1,)), ((), ())),
                              preferred_element_type=_F32) + blc_ref[...]
    xl = jnp.dot(xin, wl_ref[...], preferred_element_type=_F32) + blr_ref[...]
    xre = jnp.dot(xt, wr_ref[...], preferred_element_type=_F32) + brwe_ref[...]
    pairT = xlT[None, :, :] + xre[:, :, None]
    absT = jnp.abs(pairT)
    h_ids = jax.lax.broadcasted_iota(jnp.int32, (_HEADS, _C), 0)
    c_ids = jax.lax.broadcasted_iota(jnp.int32, (_HEADS, _C), 1) // _OUTC
    attmT = jnp.where(h_ids == c_ids, attr_ref[...], 0.0)
    attmB = jnp.broadcast_to(attmT[None], (_DT, _HEADS, _C))
    habs = jax.lax.dot_general(attmB, absT, (((2,), (1,)), ((0,), (0,))),
                               preferred_element_type=_F32)
    alin = jax.lax.dot_general(attmT, xlT, (((1,), (0,)), ((), ())),
                               preferred_element_type=_F32)
    are = jax.lax.dot_general(xre, attmT, (((1,), (1,)), ((), ())),
                              preferred_element_type=_F32)
    neg = (a_ref[0] - 1.0) * 1e30
    logits = (0.6 * (alin[None, :, :] + are[:, :, None])
              + 0.4 * habs + neg[:, None, :])
    m = jnp.max(logits, axis=2, keepdims=True)
    p = jnp.exp(logits - m)
    alpha = p / jnp.sum(p, axis=2, keepdims=True)
    agg = jnp.dot(alpha.reshape(_DT * _HEADS, _N), xl,
                  preferred_element_type=_F32).reshape(_DT, _HEADS, _C)
    sel = jnp.where(h_ids == c_ids, 1.0, 0.0).astype(_F32)
    out = jnp.sum(agg * sel[None], axis=1) + bias_ref[...]
    o_ref[0] = jnp.where(out > 0, out, jnp.exp(out) - 1.0)


def _add_body(x_ref, g_ref, o_ref):
    xb = x_ref[0]
    gr = g_ref[0, 0]
    p_ids = jax.lax.broadcasted_iota(jnp.int32, (_NW, _W), 0)
    w_ids = jax.lax.broadcasted_iota(jnp.int32, (_NW, _W), 1) // _PS
    emat = jnp.where(p_ids == w_ids, 1.0, 0.0).astype(_F32)
    wide = jnp.dot(gr, emat, preferred_element_type=_F32)
    o_ref[0] = xb + wide[:, None, :]


def _gat_layer(xin, adj, Wl, bl, Wr, br, We, att, bias):
    return pl.pallas_call(
        _gat_body,
        grid=(_B, _NT),
        in_specs=[
            pl.BlockSpec((1, _N, _C), lambda b, t: (b, 0, 0)),
            pl.BlockSpec((1, _DT, _C), lambda b, t: (b, t, 0)),
            pl.BlockSpec((1, _DT, _N), lambda b, t: (b, t, 0)),
            pl.BlockSpec((_C, _C), lambda b, t: (0, 0)),
            pl.BlockSpec((_C, 1), lambda b, t: (0, 0)),
            pl.BlockSpec((1, _C), lambda b, t: (0, 0)),
            pl.BlockSpec((_C, _C), lambda b, t: (0, 0)),
            pl.BlockSpec((1, _C), lambda b, t: (0, 0)),
            pl.BlockSpec((1, _C), lambda b, t: (0, 0)),
            pl.BlockSpec((1, _C), lambda b, t: (0, 0)),
        ],
        out_specs=pl.BlockSpec((1, _DT, _C), lambda b, t: (b, t, 0)),
        out_shape=jax.ShapeDtypeStruct((_B, _N, _C), _F32),
    )(xin, xin, adj, Wl, bl.reshape(_C, 1), bl.reshape(1, _C), Wr,
      (br + We.reshape(-1)).reshape(1, _C), att.reshape(1, _C),
      bias.reshape(1, _C))


def kernel(x, Wl1, bl1, Wr1, br1, We1, att1, bias1,
           Wl2, bl2, Wr2, br2, We2, att2, bias2):
    pool = pl.pallas_call(
        _pool_body,
        grid=(_B, _NH),
        in_specs=[pl.BlockSpec((1, _C, _PS, _W), lambda b, h: (b, 0, h, 0))],
        out_specs=pl.BlockSpec((1, 1, _C, _NW), lambda b, h: (b, h, 0, 0)),
        out_shape=jax.ShapeDtypeStruct((_B, _NH, _C, _NW), _F32),
    )(x)
    nf = pool.transpose(0, 1, 3, 2).reshape(_B, _N, _C)

    adj = pl.pallas_call(
        _adj_body,
        grid=(_B,),
        in_specs=[pl.BlockSpec((1, _N, _C), lambda b: (b, 0, 0))],
        out_specs=pl.BlockSpec((1, _N, _N), lambda b: (b, 0, 0)),
        out_shape=jax.ShapeDtypeStruct((_B, _N, _N), _F32),
    )(nf)

    h1 = _gat_layer(nf, adj, Wl1, bl1, Wr1, br1, We1, att1, bias1)
    h2 = _gat_layer(h1, adj, Wl2, bl2, Wr2, br2, We2, att2, bias2)

    g = h2.reshape(_B, _NH, _NW, _C).transpose(0, 1, 3, 2)

    return pl.pallas_call(
        _add_body,
        grid=(_B, _NH),
        in_specs=[
            pl.BlockSpec((1, _C, _PS, _W), lambda b, h: (b, 0, h, 0)),
            pl.BlockSpec((1, 1, _C, _NW), lambda b, h: (b, h, 0, 0)),
        ],
        out_specs=pl.BlockSpec((1, _C, _PS, _W), lambda b, h: (b, 0, h, 0)),
        out_shape=jax.ShapeDtypeStruct((_B, _C, _H, _W), _F32),
    )(x, g)


# speedup vs baseline: 3.7428x; 1.2700x over previous
import jax
import jax.numpy as jnp
from jax.experimental import pallas as pl

_B, _C, _H, _W = 2, 128, 256, 256
_PS = 16
_NH = _H // _PS
_NW = _W // _PS
_N = _NH * _NW
_HEADS = 8
_OUTC = _C // _HEADS
_THR = 0.5
_DT = 16
_NT = _N // _DT
_F32 = jnp.float32


def _pool_body(x_ref, o_ref):
    xb = x_ref[0]
    s = jnp.sum(xb, axis=1)
    w_ids = jax.lax.broadcasted_iota(jnp.int32, (_W, _NW), 0) // _PS
    p_ids = jax.lax.broadcasted_iota(jnp.int32, (_W, _NW), 1)
    pmat = jnp.where(w_ids == p_ids, 1.0 / (_PS * _PS), 0.0).astype(_F32)
    o_ref[0] = jax.lax.dot_general(pmat, s, (((0,), (1,)), ((), ())),
                                   preferred_element_type=_F32)


def _graph_layer(xin, neg_full, wl, blc, blr, wr, brwe, attr):
    xlT = jax.lax.dot_general(wl, xin, (((0,), (1,)), ((), ())),
                              preferred_element_type=_F32) + blc
    xl = jnp.dot(xin, wl, preferred_element_type=_F32) + blr
    xre_all = jnp.dot(xin, wr, preferred_element_type=_F32) + brwe
    h_ids = jax.lax.broadcasted_iota(jnp.int32, (_HEADS, _C), 0)
    c_ids = jax.lax.broadcasted_iota(jnp.int32, (_HEADS, _C), 1) // _OUTC
    attmT = jnp.where(h_ids == c_ids, attr, 0.0)
    attmB = jnp.broadcast_to(attmT[None], (_DT, _HEADS, _C))
    sel = jnp.where(h_ids == c_ids, 1.0, 0.0).astype(_F32)
    alin = jax.lax.dot_general(attmT, xlT, (((1,), (0,)), ((), ())),
                               preferred_element_type=_F32)
    outs = []
    for t in range(_NT):
        xre = xre_all[t * _DT:(t + 1) * _DT]
        pairT = xlT[None, :, :] + xre[:, :, None]
        absT = jnp.abs(pairT)
        habs = jax.lax.dot_general(attmB, absT, (((2,), (1,)), ((0,), (0,))),
                                   preferred_element_type=_F32)
        are = jax.lax.dot_general(xre, attmT, (((1,), (1,)), ((), ())),
                                  preferred_element_type=_F32)
        neg = neg_full[t * _DT:(t + 1) * _DT]
        logits = (0.6 * (alin[None, :, :] + are[:, :, None])
                  + 0.4 * habs + neg[:, None, :])
        m = jnp.max(logits, axis=2, keepdims=True)
        p = jnp.exp(logits - m)
        alpha = p / jnp.sum(p, axis=2, keepdims=True)
        agg = jnp.dot(alpha.reshape(_DT * _HEADS, _N), xl,
                      preferred_element_type=_F32).reshape(_DT, _HEADS, _C)
        outs.append(jnp.sum(agg * sel[None], axis=1))
    return jnp.concatenate(outs, axis=0)


def _graph_body(nf_ref,
                wl1_ref, blc1_ref, blr1_ref, wr1_ref, brwe1_ref, attr1_ref,
                bias1_ref,
                wl2_ref, blc2_ref, blr2_ref, wr2_ref, brwe2_ref, attr2_ref,
                bias2_ref, o_ref):
    nf = nf_ref[0]
    mu = jnp.mean(nf, axis=-1, keepdims=True)
    xc = nf - mu
    num = jax.lax.dot_general(xc, xc, (((1,), (1,)), ((), ())),
                              preferred_element_type=_F32)
    nrm = jnp.sqrt(jnp.sum(xc * xc, axis=-1, keepdims=True))
    outer = jax.lax.dot_general(nrm, nrm, (((1,), (1,)), ((), ())),
                                preferred_element_type=_F32)
    corr = num / (outer + 1e-8)
    neg_full = jnp.where(corr > _THR, 0.0, -1e30).astype(_F32)

    o1 = _graph_layer(nf, neg_full, wl1_ref[...], blc1_ref[...], blr1_ref[...],
                      wr1_ref[...], brwe1_ref[...], attr1_ref[...])
    o1 = o1 + bias1_ref[...]
    h1 = jnp.where(o1 > 0, o1, jnp.exp(o1) - 1.0)
    o2 = _graph_layer(h1, neg_full, wl2_ref[...], blc2_ref[...], blr2_ref[...],
                      wr2_ref[...], brwe2_ref[...], attr2_ref[...])
    o2 = o2 + bias2_ref[...]
    h2 = jnp.where(o2 > 0, o2, jnp.exp(o2) - 1.0)

    for ph in range(_NH):
        o_ref[0, ph] = h2[ph * _NW:(ph + 1) * _NW].T


def _add_body(x_ref, g_ref, o_ref):
    xb = x_ref[0]
    gr = g_ref[0, 0]
    p_ids = jax.lax.broadcasted_iota(jnp.int32, (_NW, _W), 0)
    w_ids = jax.lax.broadcasted_iota(jnp.int32, (_NW, _W), 1) // _PS
    emat = jnp.where(p_ids == w_ids, 1.0, 0.0).astype(_F32)
    wide = jnp.dot(gr, emat, preferred_element_type=_F32)
    o_ref[0] = xb + wide[:, None, :]


def kernel(x, Wl1, bl1, Wr1, br1, We1, att1, bias1,
           Wl2, bl2, Wr2, br2, We2, att2, bias2):
    nf = pl.pallas_call(
        _pool_body,
        grid=(_B, _NH),
        in_specs=[pl.BlockSpec((1, _C, _PS, _W), lambda b, h: (b, 0, h, 0))],
        out_specs=pl.BlockSpec((1, _NW, _C), lambda b, h: (b, h, 0)),
        out_shape=jax.ShapeDtypeStruct((_B, _N, _C), _F32),
    )(x)

    wspec = pl.BlockSpec((_C, _C), lambda b: (0, 0))
    rspec = pl.BlockSpec((1, _C), lambda b: (0, 0))
    cspec = pl.BlockSpec((_C, 1), lambda b: (0, 0))
    g = pl.pallas_call(
        _graph_body,
        grid=(_B,),
        in_specs=[
            pl.BlockSpec((1, _N, _C), lambda b: (b, 0, 0)),
            wspec, cspec, rspec, wspec, rspec, rspec, rspec,
            wspec, cspec, rspec, wspec, rspec, rspec, rspec,
        ],
        out_specs=pl.BlockSpec((1, _NH, _C, _NW), lambda b: (b, 0, 0, 0)),
        out_shape=jax.ShapeDtypeStruct((_B, _NH, _C, _NW), _F32),
    )(nf,
      Wl1, bl1.reshape(_C, 1), bl1.reshape(1, _C), Wr1,
      (br1 + We1.reshape(-1)).reshape(1, _C), att1.reshape(1, _C),
      bias1.reshape(1, _C),
      Wl2, bl2.reshape(_C, 1), bl2.reshape(1, _C), Wr2,
      (br2 + We2.reshape(-1)).reshape(1, _C), att2.reshape(1, _C),
      bias2.reshape(1, _C))

    return pl.pallas_call(
        _add_body,
        grid=(_B, _NH),
        in_specs=[
            pl.BlockSpec((1, _C, _PS, _W), lambda b, h: (b, 0, h, 0)),
            pl.BlockSpec((1, 1, _C, _NW), lambda b, h: (b, h, 0, 0)),
        ],
        out_specs=pl.BlockSpec((1, _C, _PS, _W), lambda b, h: (b, 0, h, 0)),
        out_shape=jax.ShapeDtypeStruct((_B, _C, _H, _W), _F32),
    )(x, g)


# speedup vs baseline: 4.5713x; 1.2213x over previous
import jax
import jax.numpy as jnp
from jax.experimental import pallas as pl

_B, _C, _H, _W = 2, 128, 256, 256
_PS = 16
_NH = _H // _PS
_NW = _W // _PS
_N = _NH * _NW
_HEADS = 8
_OUTC = _C // _HEADS
_THR = 0.5
_DT = 32
_NT = _N // _DT
_RB = 32
_NRB = _H // _RB
_F32 = jnp.float32


def _pool_body(x_ref, o_ref):
    xb = x_ref[0]
    w_ids = jax.lax.broadcasted_iota(jnp.int32, (_W, _NW), 0) // _PS
    p_ids = jax.lax.broadcasted_iota(jnp.int32, (_W, _NW), 1)
    pmat = jnp.where(w_ids == p_ids, 1.0 / (_PS * _PS), 0.0).astype(_F32)
    for r in range(_RB // _PS):
        s = jnp.sum(xb[:, r * _PS:(r + 1) * _PS, :], axis=1)
        o_ref[0, r * _NW:(r + 1) * _NW] = jax.lax.dot_general(
            pmat, s, (((0,), (1,)), ((), ())), preferred_element_type=_F32)


def _graph_layer(xin, neg_full, wl, blc, blr, wr, brwe, attr):
    xlT = jax.lax.dot_general(wl, xin, (((0,), (1,)), ((), ())),
                              preferred_element_type=_F32) + blc
    xl = jnp.dot(xin, wl, preferred_element_type=_F32) + blr
    xre_all = jnp.dot(xin, wr, preferred_element_type=_F32) + brwe
    h_ids = jax.lax.broadcasted_iota(jnp.int32, (_HEADS, _C), 0)
    c_ids = jax.lax.broadcasted_iota(jnp.int32, (_HEADS, _C), 1) // _OUTC
    attmT = jnp.where(h_ids == c_ids, attr, 0.0)
    attmB = jnp.broadcast_to(attmT[None], (_DT, _HEADS, _C))
    sel = jnp.where(h_ids == c_ids, 1.0, 0.0).astype(_F32)
    alin = jax.lax.dot_general(attmT, xlT, (((1,), (0,)), ((), ())),
                               preferred_element_type=_F32)
    outs = []
    for t in range(_NT):
        xre = xre_all[t * _DT:(t + 1) * _DT]
        pairT = xlT[None, :, :] + xre[:, :, None]
        absT = jnp.abs(pairT)
        habs = jax.lax.dot_general(attmB, absT, (((2,), (1,)), ((0,), (0,))),
                                   preferred_element_type=_F32)
        are = jax.lax.dot_general(xre, attmT, (((1,), (1,)), ((), ())),
                                  preferred_element_type=_F32)
        neg = neg_full[t * _DT:(t + 1) * _DT]
        logits = (0.6 * (alin[None, :, :] + are[:, :, None])
                  + 0.4 * habs + neg[:, None, :])
        m = jnp.max(logits, axis=2, keepdims=True)
        p = jnp.exp(logits - m)
        alpha = p / jnp.sum(p, axis=2, keepdims=True)
        agg = jnp.dot(alpha.reshape(_DT * _HEADS, _N), xl,
                      preferred_element_type=_F32).reshape(_DT, _HEADS, _C)
        outs.append(jnp.sum(agg * sel[None], axis=1))
    return jnp.concatenate(outs, axis=0)


def _graph_body(nf_ref,
                wl1_ref, blc1_ref, blr1_ref, wr1_ref, brwe1_ref, attr1_ref,
                bias1_ref,
                wl2_ref, blc2_ref, blr2_ref, wr2_ref, brwe2_ref, attr2_ref,
                bias2_ref, o_ref):
    nf = nf_ref[0]
    mu = jnp.mean(nf, axis=-1, keepdims=True)
    xc = nf - mu
    num = jax.lax.dot_general(xc, xc, (((1,), (1,)), ((), ())),
                              preferred_element_type=_F32)
    nrm = jnp.sqrt(jnp.sum(xc * xc, axis=-1, keepdims=True))
    outer = jax.lax.dot_general(nrm, nrm, (((1,), (1,)), ((), ())),
                                preferred_element_type=_F32)
    corr = num / (outer + 1e-8)
    neg_full = jnp.where(corr > _THR, 0.0, -1e30).astype(_F32)

    o1 = _graph_layer(nf, neg_full, wl1_ref[...], blc1_ref[...], blr1_ref[...],
                      wr1_ref[...], brwe1_ref[...], attr1_ref[...])
    o1 = o1 + bias1_ref[...]
    h1 = jnp.where(o1 > 0, o1, jnp.exp(o1) - 1.0)
    o2 = _graph_layer(h1, neg_full, wl2_ref[...], blc2_ref[...], blr2_ref[...],
                      wr2_ref[...], brwe2_ref[...], attr2_ref[...])
    o2 = o2 + bias2_ref[...]
    h2 = jnp.where(o2 > 0, o2, jnp.exp(o2) - 1.0)

    for ph in range(_NH):
        o_ref[0, ph] = h2[ph * _NW:(ph + 1) * _NW].T


def _add_body(x_ref, g_ref, o_ref):
    xb = x_ref[0]
    p_ids = jax.lax.broadcasted_iota(jnp.int32, (_NW, _W), 0)
    w_ids = jax.lax.broadcasted_iota(jnp.int32, (_NW, _W), 1) // _PS
    emat = jnp.where(p_ids == w_ids, 1.0, 0.0).astype(_F32)
    for r in range(_RB // _PS):
        gr = g_ref[0, r]
        wide = jnp.dot(gr, emat, preferred_element_type=_F32)
        o_ref[0, :, r * _PS:(r + 1) * _PS, :] = (
            xb[:, r * _PS:(r + 1) * _PS, :] + wide[:, None, :])


def kernel(x, Wl1, bl1, Wr1, br1, We1, att1, bias1,
           Wl2, bl2, Wr2, br2, We2, att2, bias2):
    nf = pl.pallas_call(
        _pool_body,
        grid=(_B, _NRB),
        in_specs=[pl.BlockSpec((1, _C, _RB, _W), lambda b, h: (b, 0, h, 0))],
        out_specs=pl.BlockSpec((1, (_RB // _PS) * _NW, _C), lambda b, h: (b, h, 0)),
        out_shape=jax.ShapeDtypeStruct((_B, _N, _C), _F32),
    )(x)

    wspec = pl.BlockSpec((_C, _C), lambda b: (0, 0))
    rspec = pl.BlockSpec((1, _C), lambda b: (0, 0))
    cspec = pl.BlockSpec((_C, 1), lambda b: (0, 0))
    g = pl.pallas_call(
        _graph_body,
        grid=(_B,),
        in_specs=[
            pl.BlockSpec((1, _N, _C), lambda b: (b, 0, 0)),
            wspec, cspec, rspec, wspec, rspec, rspec, rspec,
            wspec, cspec, rspec, wspec, rspec, rspec, rspec,
        ],
        out_specs=pl.BlockSpec((1, _NH, _C, _NW), lambda b: (b, 0, 0, 0)),
        out_shape=jax.ShapeDtypeStruct((_B, _NH, _C, _NW), _F32),
    )(nf,
      Wl1, bl1.reshape(_C, 1), bl1.reshape(1, _C), Wr1,
      (br1 + We1.reshape(-1)).reshape(1, _C), att1.reshape(1, _C),
      bias1.reshape(1, _C),
      Wl2, bl2.reshape(_C, 1), bl2.reshape(1, _C), Wr2,
      (br2 + We2.reshape(-1)).reshape(1, _C), att2.reshape(1, _C),
      bias2.reshape(1, _C))

    return pl.pallas_call(
        _add_body,
        grid=(_B, _NRB),
        in_specs=[
            pl.BlockSpec((1, _C, _RB, _W), lambda b, h: (b, 0, h, 0)),
            pl.BlockSpec((1, _RB // _PS, _C, _NW), lambda b, h: (b, h, 0, 0)),
        ],
        out_specs=pl.BlockSpec((1, _C, _RB, _W), lambda b, h: (b, 0, h, 0)),
        out_shape=jax.ShapeDtypeStruct((_B, _C, _H, _W), _F32),
    )(x, g)


# speedup vs baseline: 5.0208x; 1.0983x over previous
import jax
import jax.numpy as jnp
from jax.experimental import pallas as pl

_B, _C, _H, _W = 2, 128, 256, 256
_PS = 16
_NH = _H // _PS
_NW = _W // _PS
_N = _NH * _NW
_HEADS = 8
_OUTC = _C // _HEADS
_THR = 0.5
_DT = 64
_NT = _N // _DT
_RB = 64
_NRB = _H // _RB
_F32 = jnp.float32


def _pool_body(x_ref, o_ref):
    xb = x_ref[0]
    w_ids = jax.lax.broadcasted_iota(jnp.int32, (_W, _NW), 0) // _PS
    p_ids = jax.lax.broadcasted_iota(jnp.int32, (_W, _NW), 1)
    pmat = jnp.where(w_ids == p_ids, 1.0 / (_PS * _PS), 0.0).astype(_F32)
    for r in range(_RB // _PS):
        s = jnp.sum(xb[:, r * _PS:(r + 1) * _PS, :], axis=1)
        o_ref[0, r * _NW:(r + 1) * _NW] = jax.lax.dot_general(
            pmat, s, (((0,), (1,)), ((), ())), preferred_element_type=_F32)


def _graph_layer(xin, neg_full, wl, blc, blr, wr, brwe, attr):
    xlT = jax.lax.dot_general(wl, xin, (((0,), (1,)), ((), ())),
                              preferred_element_type=_F32) + blc
    xl = jnp.dot(xin, wl, preferred_element_type=_F32) + blr
    xre_all = jnp.dot(xin, wr, preferred_element_type=_F32) + brwe
    h_ids = jax.lax.broadcasted_iota(jnp.int32, (_HEADS, _C), 0)
    c_ids = jax.lax.broadcasted_iota(jnp.int32, (_HEADS, _C), 1) // _OUTC
    attmT = jnp.where(h_ids == c_ids, attr, 0.0)
    attmB = jnp.broadcast_to(attmT[None], (_DT, _HEADS, _C))
    sel = jnp.where(h_ids == c_ids, 1.0, 0.0).astype(_F32)
    alin = jax.lax.dot_general(attmT, xlT, (((1,), (0,)), ((), ())),
                               preferred_element_type=_F32)
    outs = []
    for t in range(_NT):
        xre = xre_all[t * _DT:(t + 1) * _DT]
        pairT = xlT[None, :, :] + xre[:, :, None]
        absT = jnp.abs(pairT)
        habs = jax.lax.dot_general(attmB, absT, (((2,), (1,)), ((0,), (0,))),
                                   preferred_element_type=_F32)
        are = jax.lax.dot_general(xre, attmT, (((1,), (1,)), ((), ())),
                                  preferred_element_type=_F32)
        neg = neg_full[t * _DT:(t + 1) * _DT]
        logits = (0.6 * (alin[None, :, :] + are[:, :, None])
                  + 0.4 * habs + neg[:, None, :])
        m = jnp.max(logits, axis=2, keepdims=True)
        p = jnp.exp(logits - m)
        alpha = p / jnp.sum(p, axis=2, keepdims=True)
        agg = jnp.dot(alpha.reshape(_DT * _HEADS, _N), xl,
                      preferred_element_type=_F32).reshape(_DT, _HEADS, _C)
        outs.append(jnp.sum(agg * sel[None], axis=1))
    return jnp.concatenate(outs, axis=0)


def _graph_body(nf_ref,
                wl1_ref, blc1_ref, blr1_ref, wr1_ref, brwe1_ref, attr1_ref,
                bias1_ref,
                wl2_ref, blc2_ref, blr2_ref, wr2_ref, brwe2_ref, attr2_ref,
                bias2_ref, o_ref):
    nf = nf_ref[0]
    mu = jnp.mean(nf, axis=-1, keepdims=True)
    xc = nf - mu
    num = jax.lax.dot_general(xc, xc, (((1,), (1,)), ((), ())),
                              preferred_element_type=_F32)
    nrm = jnp.sqrt(jnp.sum(xc * xc, axis=-1, keepdims=True))
    outer = jax.lax.dot_general(nrm, nrm, (((1,), (1,)), ((), ())),
                                preferred_element_type=_F32)
    corr = num / (outer + 1e-8)
    neg_full = jnp.where(corr > _THR, 0.0, -1e30).astype(_F32)

    o1 = _graph_layer(nf, neg_full, wl1_ref[...], blc1_ref[...], blr1_ref[...],
                      wr1_ref[...], brwe1_ref[...], attr1_ref[...])
    o1 = o1 + bias1_ref[...]
    h1 = jnp.where(o1 > 0, o1, jnp.exp(o1) - 1.0)
    o2 = _graph_layer(h1, neg_full, wl2_ref[...], blc2_ref[...], blr2_ref[...],
                      wr2_ref[...], brwe2_ref[...], attr2_ref[...])
    o2 = o2 + bias2_ref[...]
    h2 = jnp.where(o2 > 0, o2, jnp.exp(o2) - 1.0)

    for ph in range(_NH):
        o_ref[0, ph] = h2[ph * _NW:(ph + 1) * _NW].T


def _add_body(x_ref, g_ref, o_ref):
    xb = x_ref[0]
    p_ids = jax.lax.broadcasted_iota(jnp.int32, (_NW, _W), 0)
    w_ids = jax.lax.broadcasted_iota(jnp.int32, (_NW, _W), 1) // _PS
    emat = jnp.where(p_ids == w_ids, 1.0, 0.0).astype(_F32)
    for r in range(_RB // _PS):
        gr = g_ref[0, r]
        wide = jnp.dot(gr, emat, preferred_element_type=_F32)
        o_ref[0, :, r * _PS:(r + 1) * _PS, :] = (
            xb[:, r * _PS:(r + 1) * _PS, :] + wide[:, None, :])


def kernel(x, Wl1, bl1, Wr1, br1, We1, att1, bias1,
           Wl2, bl2, Wr2, br2, We2, att2, bias2):
    nf = pl.pallas_call(
        _pool_body,
        grid=(_B, _NRB),
        in_specs=[pl.BlockSpec((1, _C, _RB, _W), lambda b, h: (b, 0, h, 0))],
        out_specs=pl.BlockSpec((1, (_RB // _PS) * _NW, _C), lambda b, h: (b, h, 0)),
        out_shape=jax.ShapeDtypeStruct((_B, _N, _C), _F32),
    )(x)

    wspec = pl.BlockSpec((_C, _C), lambda b: (0, 0))
    rspec = pl.BlockSpec((1, _C), lambda b: (0, 0))
    cspec = pl.BlockSpec((_C, 1), lambda b: (0, 0))
    g = pl.pallas_call(
        _graph_body,
        grid=(_B,),
        in_specs=[
            pl.BlockSpec((1, _N, _C), lambda b: (b, 0, 0)),
            wspec, cspec, rspec, wspec, rspec, rspec, rspec,
            wspec, cspec, rspec, wspec, rspec, rspec, rspec,
        ],
        out_specs=pl.BlockSpec((1, _NH, _C, _NW), lambda b: (b, 0, 0, 0)),
        out_shape=jax.ShapeDtypeStruct((_B, _NH, _C, _NW), _F32),
    )(nf,
      Wl1, bl1.reshape(_C, 1), bl1.reshape(1, _C), Wr1,
      (br1 + We1.reshape(-1)).reshape(1, _C), att1.reshape(1, _C),
      bias1.reshape(1, _C),
      Wl2, bl2.reshape(_C, 1), bl2.reshape(1, _C), Wr2,
      (br2 + We2.reshape(-1)).reshape(1, _C), att2.reshape(1, _C),
      bias2.reshape(1, _C))

    return pl.pallas_call(
        _add_body,
        grid=(_B, _NRB),
        in_specs=[
            pl.BlockSpec((1, _C, _RB, _W), lambda b, h: (b, 0, h, 0)),
            pl.BlockSpec((1, _RB // _PS, _C, _NW), lambda b, h: (b, h, 0, 0)),
        ],
        out_specs=pl.BlockSpec((1, _C, _RB, _W), lambda b, h: (b, 0, h, 0)),
        out_shape=jax.ShapeDtypeStruct((_B, _C, _H, _W), _F32),
    )(x, g)


# speedup vs baseline: 5.1872x; 1.0331x over previous
import jax
import jax.numpy as jnp
from jax.experimental import pallas as pl
from jax.experimental.pallas import tpu as pltpu

_B, _C, _H, _W = 2, 128, 256, 256
_PS = 16
_NH = _H // _PS
_NW = _W // _PS
_N = _NH * _NW
_HEADS = 8
_OUTC = _C // _HEADS
_THR = 0.5
_DT = 64
_NT = _N // _DT
_RB = 64
_NRB = _H // _RB
_RPP = _RB // _PS
_F32 = jnp.float32


def _graph_layer(xin, neg_full, wl, blc, blr, wr, brwe, attr):
    xlT = jax.lax.dot_general(wl, xin, (((0,), (1,)), ((), ())),
                              preferred_element_type=_F32) + blc
    xl = jnp.dot(xin, wl, preferred_element_type=_F32) + blr
    xre_all = jnp.dot(xin, wr, preferred_element_type=_F32) + brwe
    h_ids = jax.lax.broadcasted_iota(jnp.int32, (_HEADS, _C), 0)
    c_ids = jax.lax.broadcasted_iota(jnp.int32, (_HEADS, _C), 1) // _OUTC
    attmT = jnp.where(h_ids == c_ids, attr, 0.0)
    attmB = jnp.broadcast_to(attmT[None], (_DT, _HEADS, _C))
    sel = jnp.where(h_ids == c_ids, 1.0, 0.0).astype(_F32)
    alin = jax.lax.dot_general(attmT, xlT, (((1,), (0,)), ((), ())),
                               preferred_element_type=_F32)
    outs = []
    for t in range(_NT):
        xre = xre_all[t * _DT:(t + 1) * _DT]
        pairT = xlT[None, :, :] + xre[:, :, None]
        absT = jnp.abs(pairT)
        habs = jax.lax.dot_general(attmB, absT, (((2,), (1,)), ((0,), (0,))),
                                   preferred_element_type=_F32)
        are = jax.lax.dot_general(xre, attmT, (((1,), (1,)), ((), ())),
                                  preferred_element_type=_F32)
        neg = neg_full[t * _DT:(t + 1) * _DT]
        logits = (0.6 * (alin[None, :, :] + are[:, :, None])
                  + 0.4 * habs + neg[:, None, :])
        m = jnp.max(logits, axis=2, keepdims=True)
        p = jnp.exp(logits - m)
        alpha = p / jnp.sum(p, axis=2, keepdims=True)
        agg = jnp.dot(alpha.reshape(_DT * _HEADS, _N), xl,
                      preferred_element_type=_F32).reshape(_DT, _HEADS, _C)
        outs.append(jnp.sum(agg * sel[None], axis=1))
    return jnp.concatenate(outs, axis=0)


def _mega_body(x_ref,
               wl1_ref, blc1_ref, blr1_ref, wr1_ref, brwe1_ref, attr1_ref,
               bias1_ref,
               wl2_ref, blc2_ref, blr2_ref, wr2_ref, brwe2_ref, attr2_ref,
               bias2_ref, o_ref, nf_s, g_s):
    h = pl.program_id(1)

    @pl.when(h < _NRB)
    def _pool():
        xb = x_ref[0]
        w_ids = jax.lax.broadcasted_iota(jnp.int32, (_W, _NW), 0) // _PS
        p_ids = jax.lax.broadcasted_iota(jnp.int32, (_W, _NW), 1)
        pmat = jnp.where(w_ids == p_ids, 1.0 / (_PS * _PS), 0.0).astype(_F32)
        for r in range(_RPP):
            s = jnp.sum(xb[:, r * _PS:(r + 1) * _PS, :], axis=1)
            rows = jax.lax.dot_general(pmat, s, (((0,), (1,)), ((), ())),
                                       preferred_element_type=_F32)
            nf_s[pl.ds(h * (_RPP * _NW) + r * _NW, _NW)] = rows

    @pl.when(h == _NRB)
    def _graph():
        nf = nf_s[...]
        mu = jnp.mean(nf, axis=-1, keepdims=True)
        xc = nf - mu
        num = jax.lax.dot_general(xc, xc, (((1,), (1,)), ((), ())),
                                  preferred_element_type=_F32)
        nrm = jnp.sqrt(jnp.sum(xc * xc, axis=-1, keepdims=True))
        outer = jax.lax.dot_general(nrm, nrm, (((1,), (1,)), ((), ())),
                                    preferred_element_type=_F32)
        corr = num / (outer + 1e-8)
        neg_full = jnp.where(corr > _THR, 0.0, -1e30).astype(_F32)

        o1 = _graph_layer(nf, neg_full, wl1_ref[...], blc1_ref[...],
                          blr1_ref[...], wr1_ref[...], brwe1_ref[...],
                          attr1_ref[...])
        o1 = o1 + bias1_ref[...]
        h1 = jnp.where(o1 > 0, o1, jnp.exp(o1) - 1.0)
        o2 = _graph_layer(h1, neg_full, wl2_ref[...], blc2_ref[...],
                          blr2_ref[...], wr2_ref[...], brwe2_ref[...],
                          attr2_ref[...])
        o2 = o2 + bias2_ref[...]
        h2 = jnp.where(o2 > 0, o2, jnp.exp(o2) - 1.0)
        for ph in range(_NH):
            g_s[ph] = h2[ph * _NW:(ph + 1) * _NW].T

    @pl.when(h > _NRB)
    def _add():
        xb = x_ref[0]
        p_ids = jax.lax.broadcasted_iota(jnp.int32, (_NW, _W), 0)
        w_ids = jax.lax.broadcasted_iota(jnp.int32, (_NW, _W), 1) // _PS
        emat = jnp.where(p_ids == w_ids, 1.0, 0.0).astype(_F32)
        hb = h - _NRB - 1
        for r in range(_RPP):
            gr = g_s[pl.ds(hb * _RPP + r, 1)][0]
            wide = jnp.dot(gr, emat, preferred_element_type=_F32)
            o_ref[0, :, r * _PS:(r + 1) * _PS, :] = (
                xb[:, r * _PS:(r + 1) * _PS, :] + wide[:, None, :])


def kernel(x, Wl1, bl1, Wr1, br1, We1, att1, bias1,
           Wl2, bl2, Wr2, br2, We2, att2, bias2):
    wspec = pl.BlockSpec((_C, _C), lambda b, h: (0, 0))
    rspec = pl.BlockSpec((1, _C), lambda b, h: (0, 0))
    cspec = pl.BlockSpec((_C, 1), lambda b, h: (0, 0))
    return pl.pallas_call(
        _mega_body,
        grid=(_B, 2 * _NRB + 1),
        in_specs=[
            pl.BlockSpec(
                (1, _C, _RB, _W),
                lambda b, h: (b, 0,
                              jnp.where(h < _NRB, h,
                                        jnp.maximum(h - _NRB - 1, 0)), 0)),
            wspec, cspec, rspec, wspec, rspec, rspec, rspec,
            wspec, cspec, rspec, wspec, rspec, rspec, rspec,
        ],
        out_specs=pl.BlockSpec(
            (1, _C, _RB, _W),
            lambda b, h: (b, 0, jnp.maximum(h - _NRB - 1, 0), 0)),
        out_shape=jax.ShapeDtypeStruct((_B, _C, _H, _W), _F32),
        scratch_shapes=[
            pltpu.VMEM((_N, _C), _F32),
            pltpu.VMEM((_NH, _C, _NW), _F32),
        ],
    )(x,
      Wl1, bl1.reshape(_C, 1), bl1.reshape(1, _C), Wr1,
      (br1 + We1.reshape(-1)).reshape(1, _C), att1.reshape(1, _C),
      bias1.reshape(1, _C),
      Wl2, bl2.reshape(_C, 1), bl2.reshape(1, _C), Wr2,
      (br2 + We2.reshape(-1)).reshape(1, _C), att2.reshape(1, _C),
      bias2.reshape(1, _C))


# speedup vs baseline: 5.2504x; 1.0122x over previous
import jax
import jax.numpy as jnp
from jax.experimental import pallas as pl
from jax.experimental.pallas import tpu as pltpu

_B, _C, _H, _W = 2, 128, 256, 256
_PS = 16
_NH = _H // _PS
_NW = _W // _PS
_N = _NH * _NW
_HEADS = 8
_OUTC = _C // _HEADS
_THR = 0.5
_DT = 128
_NT = _N // _DT
_RB = 64
_NRB = _H // _RB
_RPP = _RB // _PS
_F32 = jnp.float32


def _graph_layer(xin, neg_full, wl, blc, blr, wr, brwe, attr):
    xlT = jax.lax.dot_general(wl, xin, (((0,), (1,)), ((), ())),
                              preferred_element_type=_F32) + blc
    xl = jnp.dot(xin, wl, preferred_element_type=_F32) + blr
    xre_all = jnp.dot(xin, wr, preferred_element_type=_F32) + brwe
    h_ids = jax.lax.broadcasted_iota(jnp.int32, (_HEADS, _C), 0)
    c_ids = jax.lax.broadcasted_iota(jnp.int32, (_HEADS, _C), 1) // _OUTC
    attmT = jnp.where(h_ids == c_ids, attr, 0.0)
    attmB = jnp.broadcast_to(attmT[None], (_DT, _HEADS, _C))
    sel = jnp.where(h_ids == c_ids, 1.0, 0.0).astype(_F32)
    alin = jax.lax.dot_general(attmT, xlT, (((1,), (0,)), ((), ())),
                               preferred_element_type=_F32)
    outs = []
    for t in range(_NT):
        xre = xre_all[t * _DT:(t + 1) * _DT]
        pairT = xlT[None, :, :] + xre[:, :, None]
        absT = jnp.abs(pairT)
        habs = jax.lax.dot_general(attmB, absT, (((2,), (1,)), ((0,), (0,))),
                                   preferred_element_type=_F32)
        are = jax.lax.dot_general(xre, attmT, (((1,), (1,)), ((), ())),
                                  preferred_element_type=_F32)
        neg = neg_full[t * _DT:(t + 1) * _DT]
        logits = (0.6 * (alin[None, :, :] + are[:, :, None])
                  + 0.4 * habs + neg[:, None, :])
        m = jnp.max(logits, axis=2, keepdims=True)
        p = jnp.exp(logits - m)
        alpha = p / jnp.sum(p, axis=2, keepdims=True)
        agg = jnp.dot(alpha.reshape(_DT * _HEADS, _N), xl,
                      preferred_element_type=_F32).reshape(_DT, _HEADS, _C)
        outs.append(jnp.sum(agg * sel[None], axis=1))
    return jnp.concatenate(outs, axis=0)


def _mega_body(x_ref,
               wl1_ref, blc1_ref, blr1_ref, wr1_ref, brwe1_ref, attr1_ref,
               bias1_ref,
               wl2_ref, blc2_ref, blr2_ref, wr2_ref, brwe2_ref, attr2_ref,
               bias2_ref, o_ref, nf_s, g_s):
    h = pl.program_id(1)

    @pl.when(h < _NRB)
    def _pool():
        xb = x_ref[0]
        w_ids = jax.lax.broadcasted_iota(jnp.int32, (_W, _NW), 0) // _PS
        p_ids = jax.lax.broadcasted_iota(jnp.int32, (_W, _NW), 1)
        pmat = jnp.where(w_ids == p_ids, 1.0 / (_PS * _PS), 0.0).astype(_F32)
        for r in range(_RPP):
            s = jnp.sum(xb[:, r * _PS:(r + 1) * _PS, :], axis=1)
            rows = jax.lax.dot_general(pmat, s, (((0,), (1,)), ((), ())),
                                       preferred_element_type=_F32)
            nf_s[pl.ds(h * (_RPP * _NW) + r * _NW, _NW)] = rows

    @pl.when(h == _NRB)
    def _graph():
        nf = nf_s[...]
        mu = jnp.mean(nf, axis=-1, keepdims=True)
        xc = nf - mu
        num = jax.lax.dot_general(xc, xc, (((1,), (1,)), ((), ())),
                                  preferred_element_type=_F32)
        nrm = jnp.sqrt(jnp.sum(xc * xc, axis=-1, keepdims=True))
        outer = jax.lax.dot_general(nrm, nrm, (((1,), (1,)), ((), ())),
                                    preferred_element_type=_F32)
        corr = num / (outer + 1e-8)
        neg_full = jnp.where(corr > _THR, 0.0, -1e30).astype(_F32)

        o1 = _graph_layer(nf, neg_full, wl1_ref[...], blc1_ref[...],
                          blr1_ref[...], wr1_ref[...], brwe1_ref[...],
                          attr1_ref[...])
        o1 = o1 + bias1_ref[...]
        h1 = jnp.where(o1 > 0, o1, jnp.exp(o1) - 1.0)
        o2 = _graph_layer(h1, neg_full, wl2_ref[...], blc2_ref[...],
                          blr2_ref[...], wr2_ref[...], brwe2_ref[...],
                          attr2_ref[...])
        o2 = o2 + bias2_ref[...]
        h2 = jnp.where(o2 > 0, o2, jnp.exp(o2) - 1.0)
        for ph in range(_NH):
            g_s[ph] = h2[ph * _NW:(ph + 1) * _NW].T

    @pl.when(h > _NRB)
    def _add():
        xb = x_ref[0]
        p_ids = jax.lax.broadcasted_iota(jnp.int32, (_NW, _W), 0)
        w_ids = jax.lax.broadcasted_iota(jnp.int32, (_NW, _W), 1) // _PS
        emat = jnp.where(p_ids == w_ids, 1.0, 0.0).astype(_F32)
        hb = h - _NRB - 1
        for r in range(_RPP):
            gr = g_s[pl.ds(hb * _RPP + r, 1)][0]
            wide = jnp.dot(gr, emat, preferred_element_type=_F32)
            o_ref[0, :, r * _PS:(r + 1) * _PS, :] = (
                xb[:, r * _PS:(r + 1) * _PS, :] + wide[:, None, :])


def kernel(x, Wl1, bl1, Wr1, br1, We1, att1, bias1,
           Wl2, bl2, Wr2, br2, We2, att2, bias2):
    wspec = pl.BlockSpec((_C, _C), lambda b, h: (0, 0))
    rspec = pl.BlockSpec((1, _C), lambda b, h: (0, 0))
    cspec = pl.BlockSpec((_C, 1), lambda b, h: (0, 0))
    return pl.pallas_call(
        _mega_body,
        grid=(_B, 2 * _NRB + 1),
        in_specs=[
            pl.BlockSpec(
                (1, _C, _RB, _W),
                lambda b, h: (b, 0,
                              jnp.where(h < _NRB, h,
                                        jnp.maximum(h - _NRB - 1, 0)), 0)),
            wspec, cspec, rspec, wspec, rspec, rspec, rspec,
            wspec, cspec, rspec, wspec, rspec, rspec, rspec,
        ],
        out_specs=pl.BlockSpec(
            (1, _C, _RB, _W),
            lambda b, h: (b, 0, jnp.maximum(h - _NRB - 1, 0), 0)),
        out_shape=jax.ShapeDtypeStruct((_B, _C, _H, _W), _F32),
        scratch_shapes=[
            pltpu.VMEM((_N, _C), _F32),
            pltpu.VMEM((_NH, _C, _NW), _F32),
        ],
    )(x,
      Wl1, bl1.reshape(_C, 1), bl1.reshape(1, _C), Wr1,
      (br1 + We1.reshape(-1)).reshape(1, _C), att1.reshape(1, _C),
      bias1.reshape(1, _C),
      Wl2, bl2.reshape(_C, 1), bl2.reshape(1, _C), Wr2,
      (br2 + We2.reshape(-1)).reshape(1, _C), att2.reshape(1, _C),
      bias2.reshape(1, _C))


# speedup vs baseline: 5.6879x; 1.0833x over previous
import jax
import jax.numpy as jnp
from jax.experimental import pallas as pl
from jax.experimental.pallas import tpu as pltpu

_B, _C, _H, _W = 2, 128, 256, 256
_PS = 16
_NH = _H // _PS
_NW = _W // _PS
_N = _NH * _NW
_HEADS = 8
_OUTC = _C // _HEADS
_THR = 0.5
_DT = 128
_NT = _N // _DT
_RB = 64
_NRB = _H // _RB
_RPP = _RB // _PS
_F32 = jnp.float32
_NS = 6 * _NRB


def _layer_tile(xin, neg_full, t, wl, blc, blr, wr, brwe, attr, bias):
    xlT = jax.lax.dot_general(wl, xin, (((0,), (1,)), ((), ())),
                              preferred_element_type=_F32) + blc
    xl = jnp.dot(xin, wl, preferred_element_type=_F32) + blr
    h_ids = jax.lax.broadcasted_iota(jnp.int32, (_HEADS, _C), 0)
    c_ids = jax.lax.broadcasted_iota(jnp.int32, (_HEADS, _C), 1) // _OUTC
    attmT = jnp.where(h_ids == c_ids, attr, 0.0)
    attmB = jnp.broadcast_to(attmT[None], (_DT, _HEADS, _C))
    sel = jnp.where(h_ids == c_ids, 1.0, 0.0).astype(_F32)
    alin = jax.lax.dot_general(attmT, xlT, (((1,), (0,)), ((), ())),
                               preferred_element_type=_F32)
    xre = (jnp.dot(xin[t * _DT:(t + 1) * _DT], wr,
                   preferred_element_type=_F32) + brwe)
    pairT = xlT[None, :, :] + xre[:, :, None]
    absT = jnp.abs(pairT)
    habs = jax.lax.dot_general(attmB, absT, (((2,), (1,)), ((0,), (0,))),
                               preferred_element_type=_F32)
    are = jax.lax.dot_general(xre, attmT, (((1,), (1,)), ((), ())),
                              preferred_element_type=_F32)
    neg = neg_full[t * _DT:(t + 1) * _DT]
    logits = (0.6 * (alin[None, :, :] + are[:, :, None])
              + 0.4 * habs + neg[:, None, :])
    m = jnp.max(logits, axis=2, keepdims=True)
    p = jnp.exp(logits - m)
    alpha = p / jnp.sum(p, axis=2, keepdims=True)
    agg = jnp.dot(alpha.reshape(_DT * _HEADS, _N), xl,
                  preferred_element_type=_F32).reshape(_DT, _HEADS, _C)
    out = jnp.sum(agg * sel[None], axis=1) + bias
    return jnp.where(out > 0, out, jnp.exp(out) - 1.0)


def _mega_body(x_ref,
               wl1_ref, blc1_ref, blr1_ref, wr1_ref, brwe1_ref, attr1_ref,
               bias1_ref,
               wl2_ref, blc2_ref, blr2_ref, wr2_ref, brwe2_ref, attr2_ref,
               bias2_ref, o_ref, nf_s, neg_s, h1_s, g_s):
    s = pl.program_id(0)
    is_pool = jnp.logical_or(s < _NRB,
                             jnp.logical_and(s < 3 * _NRB, s % 2 == 0))
    is_g = jnp.logical_and(jnp.logical_and(s >= _NRB, s < 5 * _NRB),
                           s % 2 == 1)
    gsub = jnp.where(s < 3 * _NRB, (s - _NRB - 1) // 2,
                     (s - 3 * _NRB - 1) // 2)
    gb = jnp.where(s < 3 * _NRB, 0, 1)

    @pl.when(is_pool)
    def _pool():
        bsel = jnp.where(s < _NRB, 0, 1)
        rb = jnp.where(s < _NRB, s, (s - _NRB) // 2)
        xb = x_ref[0]
        w_ids = jax.lax.broadcasted_iota(jnp.int32, (_W, _NW), 0) // _PS
        p_ids = jax.lax.broadcasted_iota(jnp.int32, (_W, _NW), 1)
        pmat = jnp.where(w_ids == p_ids, 1.0 / (_PS * _PS), 0.0).astype(_F32)
        for r in range(_RPP):
            srow = jnp.sum(xb[:, r * _PS:(r + 1) * _PS, :], axis=1)
            rows = jax.lax.dot_general(pmat, srow, (((0,), (1,)), ((), ())),
                                       preferred_element_type=_F32)
            nf_s[bsel, pl.ds(rb * (_RPP * _NW) + r * _NW, _NW)] = rows

    @pl.when(jnp.logical_and(is_g, gsub == 0))
    def _g0():
        nf = nf_s[gb]
        mu = jnp.mean(nf, axis=-1, keepdims=True)
        xc = nf - mu
        num = jax.lax.dot_general(xc, xc, (((1,), (1,)), ((), ())),
                                  preferred_element_type=_F32)
        nrm = jnp.sqrt(jnp.sum(xc * xc, axis=-1, keepdims=True))
        outer = jax.lax.dot_general(nrm, nrm, (((1,), (1,)), ((), ())),
                                    preferred_element_type=_F32)
        corr = num / (outer + 1e-8)
        neg_full = jnp.where(corr > _THR, 0.0, -1e30).astype(_F32)
        neg_s[...] = neg_full
        h1_s[pl.ds(0, _DT)] = _layer_tile(
            nf, neg_full, 0, wl1_ref[...], blc1_ref[...], blr1_ref[...],
            wr1_ref[...], brwe1_ref[...], attr1_ref[...], bias1_ref[...])

    @pl.when(jnp.logical_and(is_g, gsub == 1))
    def _g1():
        nf = nf_s[gb]
        h1_s[pl.ds(_DT, _DT)] = _layer_tile(
            nf, neg_s[...], 1, wl1_ref[...], blc1_ref[...], blr1_ref[...],
            wr1_ref[...], brwe1_ref[...], attr1_ref[...], bias1_ref[...])

    def _g_l2(t):
        h1 = h1_s[...]
        h2t = _layer_tile(
            h1, neg_s[...], t, wl2_ref[...], blc2_ref[...], blr2_ref[...],
            wr2_ref[...], brwe2_ref[...], attr2_ref[...], bias2_ref[...])
        for ph in range(_DT // _NW):
            g_s[gb, t * (_DT // _NW) + ph] = h2t[ph * _NW:(ph + 1) * _NW].T

    @pl.when(jnp.logical_and(is_g, gsub == 2))
    def _g2():
        _g_l2(0)

    @pl.when(jnp.logical_and(is_g, gsub == 3))
    def _g3():
        _g_l2(1)

    @pl.when(jnp.logical_and(jnp.logical_not(is_pool),
                             jnp.logical_not(is_g)))
    def _add():
        bsel = jnp.where(s < 5 * _NRB, 0, 1)
        rb = jnp.where(s < 5 * _NRB, (s - 3 * _NRB) // 2, s - 5 * _NRB)
        xb = x_ref[0]
        p_ids = jax.lax.broadcasted_iota(jnp.int32, (_NW, _W), 0)
        w_ids = jax.lax.broadcasted_iota(jnp.int32, (_NW, _W), 1) // _PS
        emat = jnp.where(p_ids == w_ids, 1.0, 0.0).astype(_F32)
        for r in range(_RPP):
            gr = g_s[bsel, pl.ds(rb * _RPP + r, 1)][0]
            wide = jnp.dot(gr, emat, preferred_element_type=_F32)
            o_ref[0, :, r * _PS:(r + 1) * _PS, :] = (
                xb[:, r * _PS:(r + 1) * _PS, :] + wide[:, None, :])


def _x_index(s):
    b = jnp.where(s < _NRB, 0,
                  jnp.where(s < 3 * _NRB, 1,
                            jnp.where(s < 5 * _NRB, 0, 1)))
    row = jnp.where(s < _NRB, s,
                    jnp.where(s < 3 * _NRB, (s - _NRB) // 2,
                              jnp.where(s < 5 * _NRB, (s - 3 * _NRB) // 2,
                                        s - 5 * _NRB)))
    return (b, 0, row, 0)


def _o_index(s):
    b = jnp.where(s < 5 * _NRB, 0, 1)
    row = jnp.where(s < 5 * _NRB,
                    jnp.maximum((s - 3 * _NRB) // 2, 0), s - 5 * _NRB)
    return (b, 0, row, 0)


def kernel(x, Wl1, bl1, Wr1, br1, We1, att1, bias1,
           Wl2, bl2, Wr2, br2, We2, att2, bias2):
    wspec = pl.BlockSpec((_C, _C), lambda s: (0, 0))
    rspec = pl.BlockSpec((1, _C), lambda s: (0, 0))
    cspec = pl.BlockSpec((_C, 1), lambda s: (0, 0))
    return pl.pallas_call(
        _mega_body,
        grid=(_NS,),
        in_specs=[
            pl.BlockSpec((1, _C, _RB, _W), _x_index),
            wspec, cspec, rspec, wspec, rspec, rspec, rspec,
            wspec, cspec, rspec, wspec, rspec, rspec, rspec,
        ],
        out_specs=pl.BlockSpec((1, _C, _RB, _W), _o_index),
        out_shape=jax.ShapeDtypeStruct((_B, _C, _H, _W), _F32),
        scratch_shapes=[
            pltpu.VMEM((_B, _N, _C), _F32),
            pltpu.VMEM((_N, _N), _F32),
            pltpu.VMEM((_N, _C), _F32),
            pltpu.VMEM((_B, _NH, _C, _NW), _F32),
        ],
    )(x,
      Wl1, bl1.reshape(_C, 1), bl1.reshape(1, _C), Wr1,
      (br1 + We1.reshape(-1)).reshape(1, _C), att1.reshape(1, _C),
      bias1.reshape(1, _C),
      Wl2, bl2.reshape(_C, 1), bl2.reshape(1, _C), Wr2,
      (br2 + We2.reshape(-1)).reshape(1, _C), att2.reshape(1, _C),
      bias2.reshape(1, _C))
